# baseline (speedup 1.0000x reference)
import jax
import jax.numpy as jnp
from jax.experimental import pallas as pl
from jax.experimental.pallas import tpu as pltpu

DELTA = 1.0
ROW_BLOCK = 512


def _mine_kernel(q_all_ref, t_all_ref, out_ref,
                 a_ref, c_ref, tc_ref, loss_ref, d_ref):
    i = pl.program_id(0)
    nb = pl.num_programs(0)
    B, F = q_all_ref.shape
    R = ROW_BLOCK

    @pl.when(i == 0)
    def _build_augmented():
        q = q_all_ref[...]
        n = jnp.sum(q * q, axis=1, keepdims=True)
        a_ref[:, :F] = q
        a_ref[:, F:F + 1] = n
        a_ref[:, F + 1:F + 2] = jnp.ones_like(n)
        c_ref[:, :F] = -2.0 * q
        c_ref[:, F:F + 1] = jnp.ones_like(n)
        c_ref[:, F + 1:F + 2] = n
        tc_ref[...] = t_all_ref[...].reshape(B, 1)
        a0 = a_ref[pl.ds(0, R), :]
        d_ref[pl.ds(0, R), :] = jax.lax.dot_general(
            a0, c_ref[...],
            dimension_numbers=(((1,), (1,)), ((), ())),
            preferred_element_type=jnp.float32,
        )

    p = (i % 2) * R
    pn = ((i + 1) % 2) * R

    nxt = ((i + 1) % nb) * R
    a_next = a_ref[pl.ds(nxt, R), :]
    d_ref[pl.ds(pn, R), :] = jax.lax.dot_general(
        a_next, c_ref[...],
        dimension_numbers=(((1,), (1,)), ((), ())),
        preferred_element_type=jnp.float32,
    )

    d = d_ref[pl.ds(p, R), :]
    same = tc_ref[pl.ds(i * R, R), :] == t_all_ref[...]

    inf = jnp.float32(jnp.inf)
    m_pos = jnp.max(jnp.where(same, d, 0.0), axis=1, keepdims=True)
    m_neg = jnp.min(jnp.where(same, inf, d), axis=1, keepdims=True)

    loss_ref[...] = jnp.maximum(DELTA - m_pos + m_neg, 0.0)

    n_row = a_ref[pl.ds(i * R, R), F:F + 1]
    ok = (jnp.min(m_pos - 1e-3 * n_row) > 0.0) & (jnp.max(m_neg) < inf)

    @pl.when(jnp.logical_not(ok))
    def _exact_fix():
        rid_l = jax.lax.broadcasted_iota(jnp.int32, (R, 1), 0)
        diag_l = jax.lax.broadcasted_iota(jnp.int32, (R, R), 1) == rid_l
        local = d_ref[pl.ds(p, R), pl.ds(i * R, R)]
        d_ref[pl.ds(p, R), pl.ds(i * R, R)] = jnp.where(diag_l, 0.0, local)
        dcz = jnp.maximum(d_ref[pl.ds(p, R), :], 0.0)
        mp = jnp.max(jnp.where(same, dcz, 0.0), axis=1, keepdims=True)
        first_row = rid_l + i * R == 0
        d_pos = jnp.where(
            mp > 0.0, mp,
            jnp.where(first_row, dcz[:, 1:2], dcz[:, 0:1]))
        mn = jnp.min(jnp.where(same, inf, dcz), axis=1, keepdims=True)
        d_neg = jnp.where(mn < inf, mn, dcz[:, 0:1])
        loss_ref[...] = jnp.maximum(DELTA - d_pos + d_neg, 0.0)

    @pl.when(i == 0)
    def _init():
        out_ref[0] = 0.0

    out_ref[0] += jnp.sum(loss_ref[...])

    @pl.when(i == nb - 1)
    def _finish():
        out_ref[0] = out_ref[0] * (1.0 / B)


@jax.jit
def kernel(queries, targets):
    B, F = queries.shape
    t = targets.astype(jnp.int32)
    num_blocks = B // ROW_BLOCK

    loss = pl.pallas_call(
        _mine_kernel,
        grid=(num_blocks,),
        in_specs=[
            pl.BlockSpec((B, F), lambda i: (0, 0)),
            pl.BlockSpec((1, B), lambda i: (0, 0)),
        ],
        out_specs=pl.BlockSpec(memory_space=pltpu.MemorySpace.SMEM),
        out_shape=jax.ShapeDtypeStruct((1,), jnp.float32),
        scratch_shapes=[
            pltpu.VMEM((B, F + 2), jnp.float32),
            pltpu.VMEM((B, F + 2), jnp.float32),
            pltpu.VMEM((B, 1), jnp.int32),
            pltpu.VMEM((ROW_BLOCK, 1), jnp.float32),
            pltpu.VMEM((2 * ROW_BLOCK, B), jnp.float32),
        ],
        compiler_params=pltpu.CompilerParams(
            dimension_semantics=("arbitrary",),
        ),
    )(queries, t.reshape(1, B))

    return loss[0]


# speedup vs baseline: 1.7807x; 1.7807x over previous
import jax
import jax.numpy as jnp
from jax.experimental import pallas as pl
from jax.experimental.pallas import tpu as pltpu

DELTA = 1.0
ROW_BLOCK = 1024


def _mine_kernel(q_all_ref, t_all_ref, out_ref,
                 a_ref, c_ref, tc_ref, loss_ref, d_ref):
    i = pl.program_id(0)
    nb = pl.num_programs(0)
    B, F = q_all_ref.shape
    R = ROW_BLOCK

    @pl.when(i == 0)
    def _build_augmented():
        q = q_all_ref[...]
        n = jnp.sum(q * q, axis=1, keepdims=True)
        a_ref[:, :F] = q
        a_ref[:, F:F + 1] = n
        a_ref[:, F + 1:F + 2] = jnp.ones_like(n)
        c_ref[:, :F] = -2.0 * q
        c_ref[:, F:F + 1] = jnp.ones_like(n)
        c_ref[:, F + 1:F + 2] = n
        tc_ref[...] = t_all_ref[...].reshape(B, 1)

    a_row = a_ref[pl.ds(i * R, R), :]
    d_ref[...] = jax.lax.dot_general(
        a_row, c_ref[...],
        dimension_numbers=(((1,), (1,)), ((), ())),
        preferred_element_type=jnp.float32,
    )
    d = d_ref[...]

    same = tc_ref[pl.ds(i * R, R), :] == t_all_ref[...]

    inf = jnp.float32(jnp.inf)
    m_pos = jnp.max(jnp.where(same, d, 0.0), axis=1, keepdims=True)
    m_neg = jnp.min(jnp.where(same, inf, d), axis=1, keepdims=True)

    loss_ref[...] = jnp.maximum(DELTA - m_pos + m_neg, 0.0)

    n_row = a_ref[pl.ds(i * R, R), F:F + 1]
    ok = (jnp.min(m_pos - 1e-3 * n_row) > 0.0) & (jnp.max(m_neg) < inf)

    @pl.when(jnp.logical_not(ok))
    def _exact_fix():
        rid_l = jax.lax.broadcasted_iota(jnp.int32, (R, 1), 0)
        diag_l = jax.lax.broadcasted_iota(jnp.int32, (R, R), 1) == rid_l
        local = d_ref[:, pl.ds(i * R, R)]
        d_ref[:, pl.ds(i * R, R)] = jnp.where(diag_l, 0.0, local)
        dcz = jnp.maximum(d_ref[...], 0.0)
        mp = jnp.max(jnp.where(same, dcz, 0.0), axis=1, keepdims=True)
        first_row = rid_l + i * R == 0
        d_pos = jnp.where(
            mp > 0.0, mp,
            jnp.where(first_row, dcz[:, 1:2], dcz[:, 0:1]))
        mn = jnp.min(jnp.where(same, inf, dcz), axis=1, keepdims=True)
        d_neg = jnp.where(mn < inf, mn, dcz[:, 0:1])
        loss_ref[...] = jnp.maximum(DELTA - d_pos + d_neg, 0.0)

    @pl.when(i == 0)
    def _init():
        out_ref[0] = 0.0

    out_ref[0] += jnp.sum(loss_ref[...])

    @pl.when(i == nb - 1)
    def _finish():
        out_ref[0] = out_ref[0] * (1.0 / B)


@jax.jit
def kernel(queries, targets):
    B, F = queries.shape
    t = targets.astype(jnp.int32)
    num_blocks = B // ROW_BLOCK

    loss = pl.pallas_call(
        _mine_kernel,
        grid=(num_blocks,),
        in_specs=[
            pl.BlockSpec((B, F), lambda i: (0, 0)),
            pl.BlockSpec((1, B), lambda i: (0, 0)),
        ],
        out_specs=pl.BlockSpec(memory_space=pltpu.MemorySpace.SMEM),
        out_shape=jax.ShapeDtypeStruct((1,), jnp.float32),
        scratch_shapes=[
            pltpu.VMEM((B, F + 2), jnp.float32),
            pltpu.VMEM((B, F + 2), jnp.float32),
            pltpu.VMEM((B, 1), jnp.int32),
            pltpu.VMEM((ROW_BLOCK, 1), jnp.float32),
            pltpu.VMEM((ROW_BLOCK, B), jnp.float32),
        ],
        compiler_params=pltpu.CompilerParams(
            dimension_semantics=("arbitrary",),
        ),
    )(queries, t.reshape(1, B))

    return loss[0]
